# baseline (speedup 1.0000x reference)
import functools

import jax
import jax.numpy as jnp
from jax import lax
from jax.experimental import pallas as pl
from jax.experimental.pallas import tpu as pltpu
from jax.experimental.pallas import tpu_sc as plsc

F = 26
V = 100000
D = 32
B = 4096

NC = 2
NS = 16
NW = NC * NS
BPW = B // NW
RPW = BPW * F
CHUNK = 128
NCHUNK = RPW // CHUNK
GROUPS = BPW // 16

_mesh = plsc.VectorSubcoreMesh(core_axis_name="c", subcore_axis_name="s")


@functools.partial(
    pl.kernel,
    mesh=_mesh,
    compiler_params=pltpu.CompilerParams(
        needs_layout_passes=False, use_tc_tiling_on_sc=False),
    out_type=[
        jax.ShapeDtypeStruct((B,), jnp.float32),
        jax.ShapeDtypeStruct((B * F, D), jnp.float32),
    ],
    scratch_types=[
        pltpu.VMEM((NCHUNK, CHUNK), jnp.int32),
        pltpu.VMEM((RPW, D), jnp.float32),
        pltpu.VMEM((RPW,), jnp.float32),
        pltpu.VMEM((BPW,), jnp.float32),
        pltpu.SemaphoreType.DMA,
        pltpu.SemaphoreType.DMA,
        pltpu.SemaphoreType.DMA,
    ],
)
def _fm_kernel(idx_hbm, emb_hbm, bias_hbm, out1_hbm, embout_hbm,
               idx_v, rows_v, bias_v, out_v, gsem, bsem, wsem):
    wid = lax.axis_index("s") * NC + lax.axis_index("c")
    base_row = wid * RPW
    base_samp = wid * BPW

    pltpu.sync_copy(idx_hbm.at[wid], idx_v)

    iota = lax.iota(jnp.int32, 16)

    def _add_off(t, carry):
        j = t // 8
        col = (t - j * 8) * 16
        rvec = (base_row + t * 16) + iota
        fvec = lax.rem(rvec, F)
        idx_v[j, pl.ds(col, 16)] = idx_v[j, pl.ds(col, 16)] + fvec * V
        return carry

    lax.fori_loop(0, NCHUNK * 8, _add_off, 0)

    gcopies = []
    bcopies = []
    for j in range(NCHUNK):
        gcopies.append(pltpu.async_copy(
            emb_hbm.at[idx_v.at[j]], rows_v.at[pl.ds(j * CHUNK, CHUNK)], gsem))
        bcopies.append(pltpu.async_copy(
            bias_hbm.at[idx_v.at[j]], bias_v.at[pl.ds(j * CHUNK, CHUNK)], bsem))
    for c in gcopies:
        c.wait()

    wcopy = pltpu.async_copy(
        rows_v, embout_hbm.at[pl.ds(base_row, RPW)], wsem)
    for c in bcopies:
        c.wait()

    zeros_i = jnp.zeros((16,), jnp.int32)
    zeros_f = jnp.zeros((16,), jnp.float32)
    stride = iota * F

    def _group(g, carry):
        rb = g * (16 * F)
        row_idx = [stride + (rb + f) for f in range(F)]

        bias_acc = zeros_f
        for f in range(F):
            bias_acc = bias_acc + plsc.load_gather(bias_v, [row_idx[f]])

        def _lane(d, acc):
            dvec = jnp.full((16,), d, jnp.int32)
            s = zeros_f
            q = zeros_f
            for f in range(F):
                e = plsc.load_gather(rows_v, [row_idx[f], dvec])
                s = s + e
                q = q + e * e
            return acc + (s * s - q)

        acc = lax.fori_loop(0, D, _lane, zeros_f)
        out_v[pl.ds(g * 16, 16)] = bias_acc + 0.5 * acc
        return carry

    lax.fori_loop(0, GROUPS, _group, 0)

    pltpu.sync_copy(out_v, out1_hbm.at[pl.ds(base_samp, BPW)])
    wcopy.wait()


def kernel(field_indices, emb_tables, bias_tables):
    idx_flat = field_indices.reshape(NW, NCHUNK, CHUNK)
    emb_flat = emb_tables.reshape(F * V, D)
    bias_flat = bias_tables.reshape(F * V)
    out1, emb_rows = _fm_kernel(idx_flat, emb_flat, bias_flat)
    return (out1.reshape(B, 1), emb_rows.reshape(B, F, D))


# speedup vs baseline: 2.7854x; 2.7854x over previous
import functools

import jax
import jax.numpy as jnp
from jax import lax
from jax.experimental import pallas as pl
from jax.experimental.pallas import tpu as pltpu
from jax.experimental.pallas import tpu_sc as plsc

F = 26
V = 100000
D = 32
B = 4096

NC = 2
NS = 16
NW = NC * NS

CW = 3200
NCH = 32
TAIL = 32
SW31 = 768
ROWCAP = 4096 + 16

BPW = B // NW
RPW = BPW * F
CHUNK = 128
NCHUNK = RPW // CHUNK
GROUPS = BPW // 16

_mesh = plsc.VectorSubcoreMesh(core_axis_name="c", subcore_axis_name="s")


@functools.partial(
    pl.kernel,
    mesh=_mesh,
    compiler_params=pltpu.CompilerParams(
        needs_layout_passes=False, use_tc_tiling_on_sc=True),
    out_type=jax.ShapeDtypeStruct((B * F, 128), jnp.float32),
    scratch_types=[
        pltpu.VMEM((4, 8, CW), jnp.float32),
        pltpu.VMEM((4, 8, TAIL), jnp.float32),
        pltpu.VMEM((32, 128), jnp.int32),
        pltpu.VMEM((ROWCAP,), jnp.int32),
        pltpu.VMEM((16, 128), jnp.float32),
        pltpu.VMEM((16, 128), jnp.float32),
        pltpu.VMEM((16,), jnp.int32),
        pltpu.VMEM((16,), jnp.int32),
        pltpu.SemaphoreType.DMA,
        pltpu.SemaphoreType.DMA,
        pltpu.SemaphoreType.DMA,
        pltpu.SemaphoreType.DMA,
    ],
)
def _gather_kernel(idx3, embn, emb_tail, rows_out,
                   slab, tslab, idxf, plist, stage0, stage1, ridx0, ridx1,
                   slabsem, isem, ssem0, ssem1):
    w = lax.axis_index("s") * NC + lax.axis_index("c")
    c = w
    lo = c * CW
    width = jnp.minimum(V - lo, CW)
    sw = jnp.where(c == NCH - 1, SW31, CW)

    iota = lax.iota(jnp.int32, 16)
    zeros_f = jnp.zeros((16,), jnp.float32)

    stages = (stage0, stage1)
    ridxs = (ridx0, ridx1)
    ssems = (ssem0, ssem1)

    def _field(f, fcarry):
        pltpu.async_copy(idx3.at[f], idxf, isem).wait()

        @pl.when(c < NCH - 1)
        def _():
            pltpu.async_copy(
                embn.at[pl.ds(f * 4, 4), :,
                        pl.ds(pl.multiple_of(lo, 128), CW)],
                slab, slabsem)

        @pl.when(c == NCH - 1)
        def _():
            pltpu.async_copy(
                embn.at[pl.ds(f * 4, 4), :, pl.ds((NCH - 1) * CW, SW31)],
                slab.at[:, :, pl.ds(0, SW31)], slabsem).wait()
            pltpu.async_copy(
                emb_tail.at[pl.ds(f * 4, 4)], tslab, slabsem).wait()

        def _scan(gi, cnt):
            r = gi // 8
            k = gi - r * 8
            v16 = idxf[r, pl.ds(k * 16, 16)]
            b16 = gi * 16 + iota
            vloc = v16 - lo
            mask = (vloc >= 0) & (vloc < width)
            p16 = vloc * 4096 + b16
            plsc.store_compressed(plist.at[pl.ds(cnt, 16)], p16, mask=mask)
            npop = plsc.all_reduce_population_count(mask)
            return cnt + npop[0]

        n = lax.fori_loop(0, 256, _scan, jnp.int32(0))

        @pl.when(c < NCH - 1)
        def _():
            pltpu.make_async_copy(
                embn.at[pl.ds(f * 4, 4), :, pl.ds(0, CW)], slab,
                slabsem).wait()

        ng = (n + 15) // 16

        def _serve_pair(t, carry):
            for s in range(2):
                gi = t * 2 + s
                stage, ridx, ssem = stages[s], ridxs[s], ssems[s]

                @pl.when(gi < ng)
                def _():
                    @pl.when(gi >= 2)
                    def _():
                        pltpu.make_async_copy(
                            stage, rows_out.at[ridx], ssem).wait()

                    li = jnp.minimum(gi * 16 + iota, n - 1)
                    p16 = plsc.load_gather(plist, [li])
                    vloc = p16 // 4096
                    b16 = p16 - vloc * 4096
                    in_main = vloc < sw
                    vmain = jnp.minimum(vloc, sw - 1)
                    vtail = jnp.clip(vloc - sw, 0, TAIL - 1)
                    for dt in range(4):
                        dtv = jnp.full((16,), dt, jnp.int32)
                        for ds in range(8):
                            dsv = jnp.full((16,), ds, jnp.int32)
                            e16 = plsc.load_gather(slab, [dtv, dsv, vmain])
                            t16 = plsc.load_gather(tslab, [dtv, dsv, vtail])
                            e16 = jnp.where(in_main, e16, t16)
                            plsc.store_scatter(
                                stage,
                                [iota, jnp.full((16,), dt * 8 + ds,
                                                jnp.int32)],
                                e16)
                    ridx[...] = b16 * F + f
                    pltpu.async_copy(stage, rows_out.at[ridx], ssem)
            return carry

        lax.fori_loop(0, (ng + 1) // 2, _serve_pair, jnp.int32(0))

        for s in range(2):
            @pl.when(ng > s)
            def _():
                pltpu.make_async_copy(
                    stages[s], rows_out.at[ridxs[s]], ssems[s]).wait()
        return fcarry

    lax.fori_loop(0, F, _field, jnp.int32(0))


@functools.partial(
    pl.kernel,
    mesh=_mesh,
    compiler_params=pltpu.CompilerParams(
        needs_layout_passes=False, use_tc_tiling_on_sc=True),
    out_type=jax.ShapeDtypeStruct((B,), jnp.float32),
    scratch_types=[
        pltpu.VMEM((16 * F, 128), jnp.float32),
        pltpu.VMEM((16 * F, 128), jnp.float32),
        pltpu.VMEM((BPW,), jnp.float32),
        pltpu.SemaphoreType.DMA,
        pltpu.SemaphoreType.DMA,
    ],
)
def _stats_kernel(rows_in, acc_out, buf0, buf1, acc_v, sem0, sem1):
    w = lax.axis_index("s") * NC + lax.axis_index("c")
    base_row = w * RPW

    iota = lax.iota(jnp.int32, 16)
    zeros_f = jnp.zeros((16,), jnp.float32)
    bufs = (buf0, buf1)
    sems = (sem0, sem1)

    copies = []
    for g in range(2):
        copies.append(pltpu.async_copy(
            rows_in.at[pl.ds(base_row + g * (16 * F), 16 * F)],
            bufs[g], sems[g]))

    for g in range(GROUPS):
        s = g % 2
        buf = bufs[s]
        copies[g].wait()
        if g + 2 < GROUPS:
            pass

        row_idx = [iota * F + f for f in range(F)]

        def _lane(d, acc):
            dvec = jnp.full((16,), d, jnp.int32)
            sa = zeros_f
            qa = zeros_f
            for f in range(F):
                e = plsc.load_gather(buf, [row_idx[f], dvec])
                sa = sa + e
                qa = qa + e * e
            return acc + (sa * sa - qa)

        acc = lax.fori_loop(0, D, _lane, zeros_f)
        acc_v[pl.ds(g * 16, 16)] = 0.5 * acc

        if g + 2 < GROUPS:
            copies.append(pltpu.async_copy(
                rows_in.at[pl.ds(base_row + (g + 2) * (16 * F), 16 * F)],
                bufs[s], sems[s]))

    pltpu.sync_copy(acc_v, acc_out.at[pl.ds(w * BPW, BPW)])


@functools.partial(
    pl.kernel,
    mesh=_mesh,
    compiler_params=pltpu.CompilerParams(
        needs_layout_passes=False, use_tc_tiling_on_sc=False),
    out_type=jax.ShapeDtypeStruct((B,), jnp.float32),
    scratch_types=[
        pltpu.VMEM((NCHUNK, CHUNK), jnp.int32),
        pltpu.VMEM((RPW,), jnp.float32),
        pltpu.VMEM((BPW,), jnp.float32),
        pltpu.VMEM((BPW,), jnp.float32),
        pltpu.SemaphoreType.DMA,
        pltpu.SemaphoreType.DMA,
    ],
)
def _bias_kernel(idx_hbm, bias_hbm, acc_hbm, out1_hbm,
                 idx_v, bias_v, acc_v, out_v, bsem, asem):
    wid = lax.axis_index("s") * NC + lax.axis_index("c")
    base_row = wid * RPW
    base_samp = wid * BPW

    pltpu.sync_copy(idx_hbm.at[wid], idx_v)
    acc_cp = pltpu.async_copy(acc_hbm.at[pl.ds(base_samp, BPW)], acc_v, asem)

    iota = lax.iota(jnp.int32, 16)

    def _add_off(t, carry):
        j = t // 8
        col = (t - j * 8) * 16
        rvec = (base_row + t * 16) + iota
        fvec = lax.rem(rvec, F)
        idx_v[j, pl.ds(col, 16)] = idx_v[j, pl.ds(col, 16)] + fvec * V
        return carry

    lax.fori_loop(0, NCHUNK * 8, _add_off, 0)

    bcopies = []
    for j in range(NCHUNK):
        bcopies.append(pltpu.async_copy(
            bias_hbm.at[idx_v.at[j]], bias_v.at[pl.ds(j * CHUNK, CHUNK)],
            bsem))
    for cp in bcopies:
        cp.wait()
    acc_cp.wait()

    zeros_f = jnp.zeros((16,), jnp.float32)
    stride = iota * F

    def _group(g, carry):
        rb = g * (16 * F)
        bias_acc = zeros_f
        for f in range(F):
            bias_acc = bias_acc + plsc.load_gather(bias_v, [stride + rb + f])
        out_v[pl.ds(g * 16, 16)] = bias_acc + acc_v[pl.ds(g * 16, 16)]
        return carry

    lax.fori_loop(0, GROUPS, _group, 0)

    pltpu.sync_copy(out_v, out1_hbm.at[pl.ds(base_samp, BPW)])


def kernel(field_indices, emb_tables, bias_tables):
    idx3 = field_indices.T.reshape(F, 32, 128)
    embn = emb_tables.transpose(0, 2, 1).reshape(F * D // 8, 8, V)
    emb_tail = (emb_tables[:, V - TAIL:, :]
                .transpose(0, 2, 1).reshape(F * D // 8, 8, TAIL))
    rows = _gather_kernel(idx3, embn, emb_tail)
    acc = _stats_kernel(rows)
    idx_w = field_indices.reshape(NW, NCHUNK, CHUNK)
    bias_flat = bias_tables.reshape(F * V)
    out1 = _bias_kernel(idx_w, bias_flat, acc)
    embeds = rows[:, :D].reshape(B, F, D)
    return (out1.reshape(B, 1), embeds)


# speedup vs baseline: 2.9526x; 1.0600x over previous
import functools

import jax
import jax.numpy as jnp
from jax import lax
from jax.experimental import pallas as pl
from jax.experimental.pallas import tpu as pltpu
from jax.experimental.pallas import tpu_sc as plsc

F = 26
V = 100000
D = 32
B = 4096

NC = 2
NS = 16
NW = NC * NS

CW = 1664
NCH = 61
SW62 = 128
TAIL = 128
TOFF = (NCH - 1) * CW + SW62 - (V - TAIL)
SUPW = 8 * CW
LCAP = 4112

BPW = B // NW
RPW = BPW * F
CHUNK = 128
NCHUNK = RPW // CHUNK
GROUPS = BPW // 16

_mesh = plsc.VectorSubcoreMesh(core_axis_name="c", subcore_axis_name="s")


@functools.partial(
    pl.kernel,
    mesh=_mesh,
    compiler_params=pltpu.CompilerParams(
        needs_layout_passes=False, use_tc_tiling_on_sc=True),
    out_type=[
        jax.ShapeDtypeStruct((F * 64 * LCAP,), jnp.int32),
        jax.ShapeDtypeStruct((F * 64,), jnp.int32),
    ],
    scratch_types=[
        pltpu.VMEM((32, 128), jnp.int32),
        pltpu.VMEM((8 * LCAP,), jnp.int32),
        pltpu.VMEM((8 * LCAP,), jnp.int32),
        pltpu.VMEM((64,), jnp.int32),
        pltpu.SemaphoreType.DMA,
        pltpu.SemaphoreType.DMA,
    ],
)
def _bucket_kernel(idx3, lists_out, cnt_out, idxf, supl, subl, bcnt_v,
                   isem, osem):
    w = lax.axis_index("s") * NC + lax.axis_index("c")
    iota = lax.iota(jnp.int32, 16)
    zeros_i = jnp.zeros((16,), jnp.int32)

    @pl.when(w < F)
    def _():
        f = w
        pltpu.async_copy(idx3.at[f], idxf, isem).wait()
        for q in range(4):
            bcnt_v[pl.ds(q * 16, 16)] = zeros_i

        def _p1(gi, cnt):
            r = gi // 8
            k = gi - r * 8
            v16 = idxf[r, pl.ds(k * 16, 16)]
            b16 = gi * 16 + iota
            for s in range(8):
                vloc = v16 - s * SUPW
                m = (vloc >= 0) & (vloc < SUPW)
                plsc.store_compressed(
                    supl.at[pl.ds(s * LCAP + cnt[s], 16)],
                    vloc * 4096 + b16, mask=m)
                pop = plsc.all_reduce_population_count(m)
                cnt = cnt + jnp.where(iota == s, pop, 0)
            return cnt

        cnt1 = lax.fori_loop(0, 256, _p1, zeros_i)

        ocopies = []
        for s in range(8):
            ns = cnt1[s]
            nsub = 8

            def _p2(gi, cnt, s=s, ns=ns, nsub=nsub):
                li = gi * 16 + iota
                lim = jnp.minimum(li, ns - 1)
                valid = li < ns
                p = plsc.load_gather(supl, [s * LCAP + lim])
                vl = p // 4096
                b = p - vl * 4096
                for t in range(nsub):
                    vs = vl - t * CW
                    m = valid & (vs >= 0) & (vs < CW)
                    plsc.store_compressed(
                        subl.at[pl.ds(t * LCAP + cnt[t], 16)],
                        vs * 4096 + b, mask=m)
                    pop = plsc.all_reduce_population_count(m)
                    cnt = cnt + jnp.where(iota == t, pop, 0)
                return cnt

            cnt2 = lax.fori_loop(0, (ns + 15) // 16, _p2, zeros_i)
            plsc.store_scatter(bcnt_v, [s * 8 + iota], cnt2,
                               mask=iota < nsub)
            ocopies.append(pltpu.async_copy(
                subl, lists_out.at[pl.ds((f * 64 + s * 8) * LCAP, 8 * LCAP)],
                osem))
            ocopies[-1].wait()

        pltpu.sync_copy(bcnt_v, cnt_out.at[pl.ds(f * 64, 64)])


@functools.partial(
    pl.kernel,
    mesh=_mesh,
    compiler_params=pltpu.CompilerParams(
        needs_layout_passes=False, use_tc_tiling_on_sc=True),
    out_type=jax.ShapeDtypeStruct((B * F, 128), jnp.float32),
    scratch_types=[
        pltpu.VMEM((4, 8, CW), jnp.float32),
        pltpu.VMEM((4, 8, CW), jnp.float32),
        pltpu.VMEM((4, 8, TAIL), jnp.float32),
        pltpu.VMEM((LCAP,), jnp.int32),
        pltpu.VMEM((64,), jnp.int32),
        pltpu.VMEM((16, 128), jnp.float32),
        pltpu.VMEM((16, 128), jnp.float32),
        pltpu.VMEM((16,), jnp.int32),
        pltpu.VMEM((16,), jnp.int32),
        pltpu.SemaphoreType.DMA,
        pltpu.SemaphoreType.DMA,
        pltpu.SemaphoreType.DMA,
        pltpu.SemaphoreType.DMA,
        pltpu.SemaphoreType.DMA,
        pltpu.SemaphoreType.DMA,
        pltpu.SemaphoreType.DMA,
    ],
)
def _gather_kernel(embn, emb_tail, lists_in, cnt_in, rows_out,
                   slab_a, slab_b, tslab, list_v, cnt_v,
                   stage0, stage1, ridx0, ridx1,
                   sem_a, sem_b, tsem, csem, lsem, ssem0, ssem1):
    w = lax.axis_index("s") * NC + lax.axis_index("c")
    c0 = w
    c1 = w + 32
    iota = lax.iota(jnp.int32, 16)

    stages = (stage0, stage1)
    ridxs = (ridx0, ridx1)
    ssems = (ssem0, ssem1)

    def slab_issue(c, f, buf, sem):
        @pl.when(c < NCH - 1)
        def _():
            pltpu.async_copy(
                embn.at[pl.ds(f * 4, 4), :,
                        pl.ds(pl.multiple_of(c * CW, 128), CW)], buf, sem)

        @pl.when(c == NCH - 1)
        def _():
            pltpu.async_copy(
                embn.at[pl.ds(f * 4, 4), :, pl.ds((NCH - 1) * CW, SW62)],
                buf.at[:, :, pl.ds(0, SW62)], sem)

    def slab_drain(c, buf, sem):
        @pl.when(c < NCH - 1)
        def _():
            pltpu.make_async_copy(
                embn.at[pl.ds(0, 4), :, pl.ds(0, CW)], buf, sem).wait()

        @pl.when(c == NCH - 1)
        def _():
            pltpu.make_async_copy(
                embn.at[pl.ds(0, 4), :, pl.ds(0, SW62)],
                buf.at[:, :, pl.ds(0, SW62)], sem).wait()

    def getn(c):
        grp = c // 16
        vec = cnt_v[pl.ds(pl.multiple_of(grp * 16, 16), 16)]
        return jnp.sum(jnp.where(iota == c - grp * 16, vec, 0))

    def serve(c, n, f, slab):
        sw = jnp.where(c == NCH - 1, SW62, CW)
        base = (f * 64 + c) * LCAP
        pltpu.async_copy(
            lists_in.at[pl.ds(base, 512)], list_v.at[pl.ds(0, 512)],
            lsem).wait()

        @pl.when(n > 512)
        def _():
            pltpu.async_copy(
                lists_in.at[pl.ds(base + 512, 1536)],
                list_v.at[pl.ds(512, 1536)], lsem).wait()

        @pl.when(n > 2048)
        def _():
            pltpu.async_copy(
                lists_in.at[pl.ds(base + 2048, 2048)],
                list_v.at[pl.ds(2048, 2048)], lsem).wait()

        ng = (n + 15) // 16

        def _serve_pair(t, carry):
            for s in range(2):
                gi = t * 2 + s
                stage, ridx, ssem = stages[s], ridxs[s], ssems[s]

                @pl.when(gi < ng)
                def _():
                    @pl.when(gi >= 2)
                    def _():
                        pltpu.make_async_copy(
                            stage, rows_out.at[ridx], ssem).wait()

                    li = jnp.minimum(gi * 16 + iota, n - 1)
                    p16 = plsc.load_gather(list_v, [li])
                    vloc = p16 // 4096
                    b16 = p16 - vloc * 4096
                    in_main = vloc < sw
                    vmain = jnp.minimum(vloc, sw - 1)
                    vtail = jnp.clip(vloc - sw + TOFF, 0, TAIL - 1)
                    for dt in range(4):
                        dtv = jnp.full((16,), dt, jnp.int32)
                        for ds in range(8):
                            dsv = jnp.full((16,), ds, jnp.int32)
                            e16 = plsc.load_gather(slab, [dtv, dsv, vmain])
                            t16 = plsc.load_gather(tslab, [dtv, dsv, vtail])
                            e16 = jnp.where(in_main, e16, t16)
                            plsc.store_scatter(
                                stage,
                                [iota,
                                 jnp.full((16,), dt * 8 + ds, jnp.int32)],
                                e16)
                    ridx[...] = b16 * F + f
                    pltpu.async_copy(stage, rows_out.at[ridx], ssem)
            return carry

        lax.fori_loop(0, (ng + 1) // 2, _serve_pair, jnp.int32(0))

        for s in range(2):
            @pl.when(ng > s)
            def _():
                pltpu.make_async_copy(
                    stages[s], rows_out.at[ridxs[s]], ssems[s]).wait()

    slab_issue(c0, 0, slab_a, sem_a)

    def _field(f, carry):
        pltpu.async_copy(cnt_in.at[pl.ds(f * 64, 64)], cnt_v, csem).wait()
        n0 = getn(c0)
        n1 = getn(c1)
        slab_issue(c1, f, slab_b, sem_b)

        @pl.when(c1 == NCH - 1)
        def _():
            pltpu.async_copy(emb_tail.at[pl.ds(f * 4, 4)], tslab,
                             tsem).wait()

        slab_drain(c0, slab_a, sem_a)
        serve(c0, n0, f, slab_a)

        @pl.when(f < F - 1)
        def _():
            slab_issue(c0, f + 1, slab_a, sem_a)

        slab_drain(c1, slab_b, sem_b)
        serve(c1, n1, f, slab_b)
        return carry

    lax.fori_loop(0, F, _field, jnp.int32(0))


@functools.partial(
    pl.kernel,
    mesh=_mesh,
    compiler_params=pltpu.CompilerParams(
        needs_layout_passes=False, use_tc_tiling_on_sc=True),
    out_type=jax.ShapeDtypeStruct((B,), jnp.float32),
    scratch_types=[
        pltpu.VMEM((16 * F, 128), jnp.float32),
        pltpu.VMEM((16 * F, 128), jnp.float32),
        pltpu.VMEM((BPW,), jnp.float32),
        pltpu.SemaphoreType.DMA,
        pltpu.SemaphoreType.DMA,
    ],
)
def _stats_kernel(rows_in, acc_out, buf0, buf1, acc_v, sem0, sem1):
    w = lax.axis_index("s") * NC + lax.axis_index("c")
    base_row = w * RPW

    iota = lax.iota(jnp.int32, 16)
    zeros_f = jnp.zeros((16,), jnp.float32)
    bufs = (buf0, buf1)
    sems = (sem0, sem1)

    copies = []
    for g in range(2):
        copies.append(pltpu.async_copy(
            rows_in.at[pl.ds(base_row + g * (16 * F), 16 * F)],
            bufs[g], sems[g]))

    for g in range(GROUPS):
        s = g % 2
        buf = bufs[s]
        copies[g].wait()

        row_idx = [iota * F + f for f in range(F)]

        def _lane(d, acc):
            dvec = jnp.full((16,), d, jnp.int32)
            sa = zeros_f
            qa = zeros_f
            for f in range(F):
                e = plsc.load_gather(buf, [row_idx[f], dvec])
                sa = sa + e
                qa = qa + e * e
            return acc + (sa * sa - qa)

        acc = lax.fori_loop(0, D, _lane, zeros_f)
        acc_v[pl.ds(g * 16, 16)] = 0.5 * acc

        if g + 2 < GROUPS:
            copies.append(pltpu.async_copy(
                rows_in.at[pl.ds(base_row + (g + 2) * (16 * F), 16 * F)],
                bufs[s], sems[s]))

    pltpu.sync_copy(acc_v, acc_out.at[pl.ds(w * BPW, BPW)])


@functools.partial(
    pl.kernel,
    mesh=_mesh,
    compiler_params=pltpu.CompilerParams(
        needs_layout_passes=False, use_tc_tiling_on_sc=False),
    out_type=jax.ShapeDtypeStruct((B,), jnp.float32),
    scratch_types=[
        pltpu.VMEM((NCHUNK, CHUNK), jnp.int32),
        pltpu.VMEM((RPW,), jnp.float32),
        pltpu.VMEM((BPW,), jnp.float32),
        pltpu.VMEM((BPW,), jnp.float32),
        pltpu.SemaphoreType.DMA,
        pltpu.SemaphoreType.DMA,
    ],
)
def _bias_kernel(idx_hbm, bias_hbm, acc_hbm, out1_hbm,
                 idx_v, bias_v, acc_v, out_v, bsem, asem):
    wid = lax.axis_index("s") * NC + lax.axis_index("c")
    base_row = wid * RPW
    base_samp = wid * BPW

    pltpu.sync_copy(idx_hbm.at[wid], idx_v)
    acc_cp = pltpu.async_copy(acc_hbm.at[pl.ds(base_samp, BPW)], acc_v, asem)

    iota = lax.iota(jnp.int32, 16)

    def _add_off(t, carry):
        j = t // 8
        col = (t - j * 8) * 16
        rvec = (base_row + t * 16) + iota
        fvec = lax.rem(rvec, F)
        idx_v[j, pl.ds(col, 16)] = idx_v[j, pl.ds(col, 16)] + fvec * V
        return carry

    lax.fori_loop(0, NCHUNK * 8, _add_off, 0)

    bcopies = []
    for j in range(NCHUNK):
        bcopies.append(pltpu.async_copy(
            bias_hbm.at[idx_v.at[j]], bias_v.at[pl.ds(j * CHUNK, CHUNK)],
            bsem))
    for cp in bcopies:
        cp.wait()
    acc_cp.wait()

    zeros_f = jnp.zeros((16,), jnp.float32)
    stride = iota * F

    def _group(g, carry):
        rb = g * (16 * F)
        bias_acc = zeros_f
        for f in range(F):
            bias_acc = bias_acc + plsc.load_gather(bias_v, [stride + rb + f])
        out_v[pl.ds(g * 16, 16)] = bias_acc + acc_v[pl.ds(g * 16, 16)]
        return carry

    lax.fori_loop(0, GROUPS, _group, 0)

    pltpu.sync_copy(out_v, out1_hbm.at[pl.ds(base_samp, BPW)])


def kernel(field_indices, emb_tables, bias_tables):
    idx3 = field_indices.T.reshape(F, 32, 128)
    embn = emb_tables.transpose(0, 2, 1).reshape(F * D // 8, 8, V)
    emb_tail = (emb_tables[:, V - TAIL:, :]
                .transpose(0, 2, 1).reshape(F * D // 8, 8, TAIL))
    lists, cnts = _bucket_kernel(idx3)
    rows = _gather_kernel(embn, emb_tail, lists, cnts)
    acc = _stats_kernel(rows)
    idx_w = field_indices.reshape(NW, NCHUNK, CHUNK)
    bias_flat = bias_tables.reshape(F * V)
    out1 = _bias_kernel(idx_w, bias_flat, acc)
    embeds = rows[:, :D].reshape(B, F, D)
    return (out1.reshape(B, 1), embeds)
